# baseline (speedup 1.0000x reference)
import functools

import jax
import jax.numpy as jnp
from jax import lax
from jax.experimental import pallas as pl
from jax.experimental.pallas import tpu as pltpu
from jax.experimental.pallas import tpu_sc as plsc

NUM_EMBEDDINGS = 1000000
EMBEDDING_DIM = 64
TP_WORLD_SIZE = 2
NUM_EMB_PER_PART = NUM_EMBEDDINGS // TP_WORLD_SIZE
VOCAB_START = 0
VOCAB_END = NUM_EMB_PER_PART
NUM_TOKENS = 16384

NC = 2
NS = 16
NW = NC * NS
BPW = NUM_TOKENS // NW
CH = 128
NCHUNK = BPW // CH
NPACK = NUM_EMB_PER_PART // 2
PD = 2 * EMBEDDING_DIM

_mesh = plsc.VectorSubcoreMesh(core_axis_name="c", subcore_axis_name="s")


@functools.partial(
    pl.kernel,
    mesh=_mesh,
    out_type=jax.ShapeDtypeStruct((NUM_TOKENS, EMBEDDING_DIM), jnp.float32),
    scratch_types=[
        pltpu.VMEM((BPW,), jnp.int32),
        pltpu.VMEM((NCHUNK, CH), jnp.int32),
        pltpu.VMEM((BPW,), jnp.int32),
        pltpu.VMEM((BPW,), jnp.float32),
        pltpu.VMEM((CH, PD), jnp.float32),
        pltpu.VMEM((CH, EMBEDDING_DIM), jnp.float32),
        pltpu.SemaphoreType.DMA,
    ],
    compiler_params=pltpu.CompilerParams(needs_layout_passes=False),
)
def _emb_kernel(x_hbm, w_hbm, out_hbm, idx_v, p_v, h_v, fm_v, tiles_v, rows_v, sem):
    wid = lax.axis_index("s") * NC + lax.axis_index("c")
    base = wid * BPW

    pltpu.sync_copy(x_hbm.at[pl.ds(base, BPW)], idx_v)

    ones_f = jnp.full((16,), 1.0, jnp.float32)
    zeros_f = jnp.full((16,), 0.0, jnp.float32)
    zeros_i = jnp.full((16,), 0, jnp.int32)
    span = jnp.full((16,), VOCAB_END - VOCAB_START, jnp.uint32)

    for g in range(BPW // 16):
        iv = idx_v[pl.ds(g * 16, 16)]
        rel = iv - VOCAB_START
        m = plsc.bitcast(rel, jnp.uint32) < span
        clamped = jnp.where(m, rel, zeros_i)
        p_v[g // (CH // 16), pl.ds((g % (CH // 16)) * 16, 16)] = clamped >> 1
        h_v[pl.ds(g * 16, 16)] = (clamped & 1) * EMBEDDING_DIM
        fm_v[pl.ds(g * 16, 16)] = jnp.where(m, ones_f, zeros_f)

    lanes = lax.iota(jnp.int32, 16)

    def chunk_body(k, _):
        pltpu.async_copy(w_hbm.at[p_v.at[k]], tiles_v, sem).wait()

        def group_body(g, _):
            t0 = k * CH + g * 16
            qpos = jnp.full((16,), 0, jnp.int32) + g * 16 + lanes
            hv = h_v[pl.ds(t0, 16)]
            fm = fm_v[pl.ds(t0, 16)]
            for c in range(EMBEDDING_DIM):
                cv = jnp.full((16,), c, jnp.int32)
                vals = plsc.load_gather(tiles_v, [qpos, hv + cv])
                plsc.store_scatter(rows_v, [qpos, cv], vals * fm)
            return 0

        lax.fori_loop(0, CH // 16, group_body, 0)
        pltpu.sync_copy(rows_v, out_hbm.at[pl.ds(base + k * CH, CH)])
        return 0

    lax.fori_loop(0, NCHUNK, chunk_body, 0)


def kernel(x, weight):
    w2 = weight.reshape(NPACK, PD)
    return _emb_kernel(x.astype(jnp.int32), w2)


# speedup vs baseline: 1.0087x; 1.0087x over previous
import functools

import jax
import jax.numpy as jnp
from jax import lax
from jax.experimental import pallas as pl
from jax.experimental.pallas import tpu as pltpu
from jax.experimental.pallas import tpu_sc as plsc

NUM_EMBEDDINGS = 1000000
EMBEDDING_DIM = 64
TP_WORLD_SIZE = 2
NUM_EMB_PER_PART = NUM_EMBEDDINGS // TP_WORLD_SIZE
VOCAB_START = 0
VOCAB_END = NUM_EMB_PER_PART
NUM_TOKENS = 16384

NC = 2
NS = 16
NW = NC * NS
BPW = NUM_TOKENS // NW
CH = 128
NCHUNK = BPW // CH
NPACK = NUM_EMB_PER_PART // 2
PD = 2 * EMBEDDING_DIM

_mesh = plsc.VectorSubcoreMesh(core_axis_name="c", subcore_axis_name="s")


@functools.partial(
    pl.kernel,
    mesh=_mesh,
    out_type=jax.ShapeDtypeStruct((NUM_TOKENS, EMBEDDING_DIM), jnp.float32),
    scratch_types=[
        pltpu.VMEM((BPW,), jnp.int32),
        pltpu.VMEM((NCHUNK, CH), jnp.int32),
        pltpu.VMEM((BPW,), jnp.int32),
        pltpu.VMEM((BPW,), jnp.float32),
        pltpu.VMEM((CH, PD), jnp.float32),
        pltpu.VMEM((CH, EMBEDDING_DIM), jnp.float32),
        pltpu.SemaphoreType.DMA,
    ],
    compiler_params=pltpu.CompilerParams(needs_layout_passes=False),
)
def _emb_kernel(x_hbm, w_hbm, out_hbm, idx_v, p_v, h_v, fm_v, tiles_v, rows_v, sem):
    wid = lax.axis_index("s") * NC + lax.axis_index("c")
    base = wid * BPW

    pltpu.sync_copy(x_hbm.at[pl.ds(base, BPW)], idx_v)

    ones_f = jnp.full((16,), 1.0, jnp.float32)
    zeros_f = jnp.full((16,), 0.0, jnp.float32)
    zeros_i = jnp.full((16,), 0, jnp.int32)
    span = jnp.full((16,), VOCAB_END - VOCAB_START, jnp.uint32)

    for g in range(BPW // 16):
        iv = idx_v[pl.ds(g * 16, 16)]
        rel = iv - VOCAB_START
        m = plsc.bitcast(rel, jnp.uint32) < span
        clamped = jnp.where(m, rel, zeros_i)
        p_v[g // (CH // 16), pl.ds((g % (CH // 16)) * 16, 16)] = clamped >> 1
        h_v[pl.ds(g * 16, 16)] = (clamped & 1) * EMBEDDING_DIM
        fm_v[pl.ds(g * 16, 16)] = jnp.where(m, ones_f, zeros_f)

    def chunk_body(k, _):
        pltpu.async_copy(w_hbm.at[p_v.at[k]], tiles_v, sem).wait()
        for g in range(CH // 16):
            t0 = k * CH + g * 16
            hvec = h_v[pl.ds(t0, 16)]
            fmvec = fm_v[pl.ds(t0, 16)]
            for l in range(16):
                h_l = hvec[l]
                bc = jnp.full((16,), fmvec[l], jnp.float32)
                qp = g * 16 + l
                for j in range(EMBEDDING_DIM // 16):
                    seg = tiles_v[qp, pl.ds(h_l + 16 * j, 16)]
                    rows_v[qp, pl.ds(16 * j, 16)] = seg * bc
        pltpu.sync_copy(rows_v, out_hbm.at[pl.ds(base + k * CH, CH)])
        return 0

    lax.fori_loop(0, NCHUNK, chunk_body, 0)


def kernel(x, weight):
    w2 = weight.reshape(NPACK, PD)
    return _emb_kernel(x.astype(jnp.int32), w2)
